# baseline (speedup 1.0000x reference)
import functools

import jax
import jax.numpy as jnp
from jax.experimental import pallas as pl
from jax.experimental.pallas import tpu as pltpu

_CHUNK = 4096
_NEG_INF = float("-inf")


def _body(state_ref, w_ref, b_ref, g_ref, probs_ref, act_ref,
          m_ref, s_ref, bv_ref, bi_ref, *, n_actions, n_chunks):
    phase = pl.program_id(0)
    k = pl.program_id(1)
    c = probs_ref.shape[1]

    @pl.when(jnp.logical_and(phase == 0, k == 0))
    def _init():
        m_ref[...] = jnp.full_like(m_ref, _NEG_INF)
        s_ref[...] = jnp.zeros_like(s_ref)
        bv_ref[...] = jnp.full_like(bv_ref, _NEG_INF)
        bi_ref[...] = jnp.zeros_like(bi_ref)

    logits = jnp.dot(state_ref[...], w_ref[...],
                     preferred_element_type=jnp.float32) + b_ref[...]
    col = k * c + jax.lax.broadcasted_iota(jnp.int32, logits.shape, 1)
    valid = col < n_actions

    @pl.when(phase == 0)
    def _stats():
        lm = jnp.where(valid, logits, _NEG_INF)
        cmax = jnp.max(lm, axis=1, keepdims=True)
        new_m = jnp.maximum(m_ref[...], cmax)
        e = jnp.where(valid, jnp.exp(logits - new_m), 0.0)
        s_ref[...] = s_ref[...] * jnp.exp(m_ref[...] - new_m) \
            + jnp.sum(e, axis=1, keepdims=True)
        m_ref[...] = new_m

    @pl.when(phase == 1)
    def _emit():
        p = jnp.exp(logits - m_ref[...]) * (1.0 / s_ref[...])
        probs_ref[...] = p
        val = jnp.log(p + 1e-8) + g_ref[...]
        val = jnp.where(valid, val, _NEG_INF)
        cmax = jnp.max(val, axis=1, keepdims=True)
        cidx = jnp.min(jnp.where(val == cmax, col, n_actions),
                       axis=1, keepdims=True)
        upd = cmax > bv_ref[...]
        bv_ref[...] = jnp.where(upd, cmax, bv_ref[...])
        bi_ref[...] = jnp.where(upd, cidx, bi_ref[...])

        @pl.when(k == n_chunks - 1)
        def _final():
            act_ref[...] = bi_ref[...]


def kernel(state, W, b):
    batch, d_in = state.shape
    n_actions = W.shape[1]
    g = jax.random.gumbel(jax.random.key(42), (batch, n_actions), jnp.float32)
    b2 = b.reshape(1, n_actions)
    chunk = min(_CHUNK, n_actions)
    n_chunks = pl.cdiv(n_actions, chunk)

    probs, actions = pl.pallas_call(
        functools.partial(_body, n_actions=n_actions, n_chunks=n_chunks),
        grid=(2, n_chunks),
        in_specs=[
            pl.BlockSpec((batch, d_in), lambda p, k: (0, 0)),
            pl.BlockSpec((d_in, chunk), lambda p, k: (0, k)),
            pl.BlockSpec((1, chunk), lambda p, k: (0, k)),
            pl.BlockSpec((batch, chunk), lambda p, k: (0, k * p)),
        ],
        out_specs=[
            pl.BlockSpec((batch, chunk), lambda p, k: (0, k * p)),
            pl.BlockSpec((batch, 1), lambda p, k: (0, 0)),
        ],
        out_shape=[
            jax.ShapeDtypeStruct((batch, n_actions), jnp.float32),
            jax.ShapeDtypeStruct((batch, 1), jnp.int32),
        ],
        scratch_shapes=[
            pltpu.VMEM((batch, 1), jnp.float32),
            pltpu.VMEM((batch, 1), jnp.float32),
            pltpu.VMEM((batch, 1), jnp.float32),
            pltpu.VMEM((batch, 1), jnp.int32),
        ],
    )(state, W, b2, g)
    return probs, actions
